# baseline (speedup 1.0000x reference)
import functools

import jax
import jax.numpy as jnp
from jax.experimental import pallas as pl
from jax.experimental.pallas import tpu as pltpu

B, S, H, VD, E, I, P, TOPK = 2, 2048, 1024, 1024, 16, 64, 8, 2
N = B * S
BT = 512
NB = N // BT


def _ln_last(t):
    m = jnp.mean(t, axis=-1, keepdims=True)
    v = jnp.mean((t - m) ** 2, axis=-1, keepdims=True)
    return (t - m) * jax.lax.rsqrt(v + 1e-5)


def _fused_body(x_ref, wv_ref, wg_ref, bg_ref, wp_ref, w1f_ref, b1_ref,
                w2f_ref, b2_ref, alpha_ref, out_ref,
                wv_bf, wsc_bf, w1f_bf):
    f32 = jnp.float32
    bf = jnp.bfloat16

    @pl.when(pl.program_id(0) == 0)
    def _prep():
        wv_bf[...] = wv_ref[...].astype(bf)
        wp = wp_ref[...]
        nrm = jnp.sqrt(jnp.sum(wp * wp, axis=-1, keepdims=True))
        wsc_bf[:E * P, :] = (wp / (nrm + 1e-6)).astype(bf)
        wsc_bf[E * P:, :] = wg_ref[...].astype(bf)
        w1f_bf[...] = w1f_ref[...].astype(bf)

    HB = BT // 2
    xb_full = x_ref[...]
    for lo in (0, HB):
      xb = xb_full[lo:lo + HB]
      V = jax.lax.dot_general(xb.astype(bf), wv_bf[...],
                              (((1,), (1,)), ((), ())),
                              preferred_element_type=f32)
      Vb = V.astype(bf)
      big = jax.lax.dot_general(Vb, wsc_bf[...], (((1,), (1,)), ((), ())),
                                preferred_element_type=f32)
      proj = big[:, :E * P]
      r = big[:, E * P:] + bg_ref[...]
      p2 = proj * proj
      c2 = jnp.concatenate(
          [jnp.sum(p2[:, e * P:(e + 1) * P], axis=-1, keepdims=True)
           for e in range(E)], axis=-1)
      c = jnp.sqrt(c2) / jnp.sqrt(jnp.float32(P))
      sa = 1.0 / (1.0 + jnp.exp(-alpha_ref[...]))
      s = _ln_last(r) + sa * _ln_last(c)
      lane = jax.lax.broadcasted_iota(jnp.int32, (HB, E), 1)
      m1 = jnp.max(s, axis=-1, keepdims=True)
      i1 = jnp.min(jnp.where(s == m1, lane, E), axis=-1, keepdims=True)
      oh1 = (lane == i1)
      s2 = jnp.where(oh1, -jnp.inf, s)
      m2 = jnp.max(s2, axis=-1, keepdims=True)
      i2 = jnp.min(jnp.where(s2 == m2, lane, E), axis=-1, keepdims=True)
      oh2 = (lane == i2)
      t = jnp.exp(m2 - m1)
      g1 = 1.0 / (1.0 + t)
      g2 = 1.0 - g1
      combine = oh1.astype(f32) * g1 + oh2.astype(f32) * g2
      h = jax.lax.dot_general(Vb, w1f_bf[...], (((1,), (1,)), ((), ())),
                              preferred_element_type=f32) + b1_ref[...]
      a = h * (1.0 / (1.0 + jnp.exp(-h)))
      cexp = jnp.concatenate(
          [jax.lax.broadcast_in_dim(combine[:, e:e + 1], (HB, I), (0, 1))
           for e in range(E)], axis=-1)
      aw = (a * cexp).astype(bf)
      delta = jax.lax.dot_general(aw, w2f_ref[...], (((1,), (0,)), ((), ())),
                                  preferred_element_type=f32)
      b2t = jax.lax.dot_general(combine.astype(bf), b2_ref[...].astype(bf),
                                (((1,), (0,)), ((), ())),
                                preferred_element_type=f32)
      out_ref[lo:lo + HB, :] = V + delta + b2t


@jax.jit
def kernel(x, Wv, W_gate, b_gate, W1, b1, W2, b2, alpha):
    bf = jnp.bfloat16
    x2 = x.reshape(N, H)
    Wp = W1[:, :P, :].reshape(E * P, VD)
    W1f = W1.reshape(E * I, VD)
    W2f = jnp.swapaxes(W2, 1, 2).reshape(E * I, VD).astype(bf)
    b1f = b1.reshape(1, E * I)
    bg = b_gate.reshape(1, E)
    al = alpha.reshape(1, 1)
    full = lambda shp: pl.BlockSpec(shp, lambda i: (0,) * len(shp))
    out = pl.pallas_call(
        _fused_body,
        grid=(NB,),
        in_specs=[
            pl.BlockSpec((BT, H), lambda i: (i, 0)),
            full((VD, H)),
            full((E, VD)),
            full((1, E)),
            full((E * P, VD)),
            full((E * I, VD)),
            full((1, E * I)),
            full((E * I, VD)),
            full((E, VD)),
            full((1, 1)),
        ],
        out_specs=pl.BlockSpec((BT, VD), lambda i: (i, 0)),
        out_shape=jax.ShapeDtypeStruct((N, VD), jnp.float32),
        scratch_shapes=[
            pltpu.VMEM((VD, H), bf),
            pltpu.VMEM((E * P + E, VD), bf),
            pltpu.VMEM((E * I, VD), bf),
        ],
        compiler_params=pltpu.CompilerParams(
            dimension_semantics=("arbitrary",)),
    )(x2, Wv, W_gate, bg, Wp, W1f, b1f, W2f, b2, al)
    return out.reshape(B, S, VD)


# speedup vs baseline: 1.1820x; 1.1820x over previous
import functools

import jax
import jax.numpy as jnp
from jax.experimental import pallas as pl
from jax.experimental.pallas import tpu as pltpu

B, S, H, VD, E, I, P, TOPK = 2, 2048, 1024, 1024, 16, 64, 8, 2
N = B * S
BT = 1024
NB = N // BT


def _ln_last(t):
    m = jnp.mean(t, axis=-1, keepdims=True)
    v = jnp.mean((t - m) ** 2, axis=-1, keepdims=True)
    return (t - m) * jax.lax.rsqrt(v + 1e-5)


def _fused_body(x_ref, wv_ref, wg_ref, bg_ref, wp_ref, w1f_ref, b1_ref,
                w2f_ref, b2_ref, alpha_ref, out_ref,
                wv_bf, wsc_bf, w1f_bf):
    f32 = jnp.float32
    bf = jnp.bfloat16

    @pl.when(pl.program_id(0) == 0)
    def _prep():
        wv_bf[...] = wv_ref[...].astype(bf)
        wp = wp_ref[...]
        nrm = jnp.sqrt(jnp.sum(wp * wp, axis=-1, keepdims=True))
        wsc_bf[:E * P, :] = (wp / (nrm + 1e-6)).astype(bf)
        wsc_bf[E * P:, :] = wg_ref[...].astype(bf)
        w1f_bf[...] = w1f_ref[...].astype(bf)

    xb = x_ref[...]
    V = jax.lax.dot_general(xb.astype(bf), wv_bf[...],
                            (((1,), (1,)), ((), ())),
                            preferred_element_type=f32)
    Vb = V.astype(bf)
    big = jax.lax.dot_general(Vb, wsc_bf[...], (((1,), (1,)), ((), ())),
                              preferred_element_type=f32)
    proj = big[:, :E * P]
    r = big[:, E * P:] + bg_ref[...]
    p2 = proj * proj
    c2 = jnp.concatenate(
        [jnp.sum(p2[:, e * P:(e + 1) * P], axis=-1, keepdims=True)
         for e in range(E)], axis=-1)
    c = jnp.sqrt(c2) / jnp.sqrt(jnp.float32(P))
    sa = 1.0 / (1.0 + jnp.exp(-alpha_ref[...]))
    s = _ln_last(r) + sa * _ln_last(c)
    lane = jax.lax.broadcasted_iota(jnp.int32, (BT, E), 1)
    m1 = jnp.max(s, axis=-1, keepdims=True)
    i1 = jnp.min(jnp.where(s == m1, lane, E), axis=-1, keepdims=True)
    oh1 = (lane == i1)
    s2 = jnp.where(oh1, -jnp.inf, s)
    m2 = jnp.max(s2, axis=-1, keepdims=True)
    i2 = jnp.min(jnp.where(s2 == m2, lane, E), axis=-1, keepdims=True)
    oh2 = (lane == i2)
    t = jnp.exp(m2 - m1)
    g1 = 1.0 / (1.0 + t)
    g2 = 1.0 - g1
    combine = oh1.astype(f32) * g1 + oh2.astype(f32) * g2
    h = jax.lax.dot_general(Vb, w1f_bf[...], (((1,), (1,)), ((), ())),
                            preferred_element_type=f32) + b1_ref[...]
    a = h * (1.0 / (1.0 + jnp.exp(-h)))
    cexp = jnp.concatenate(
        [jax.lax.broadcast_in_dim(combine[:, e:e + 1], (BT, I), (0, 1))
         for e in range(E)], axis=-1)
    aw = (a * cexp).astype(bf)
    delta = jax.lax.dot_general(aw, w2f_ref[...], (((1,), (0,)), ((), ())),
                                preferred_element_type=f32)
    b2t = jax.lax.dot_general(combine.astype(bf), b2_ref[...].astype(bf),
                              (((1,), (0,)), ((), ())),
                              preferred_element_type=f32)
    out_ref[...] = V + delta + b2t


@jax.jit
def kernel(x, Wv, W_gate, b_gate, W1, b1, W2, b2, alpha):
    bf = jnp.bfloat16
    x2 = x.reshape(N, H)
    Wp = W1[:, :P, :].reshape(E * P, VD)
    W1f = W1.reshape(E * I, VD)
    W2f = jnp.swapaxes(W2, 1, 2).reshape(E * I, VD).astype(bf)
    b1f = b1.reshape(1, E * I)
    bg = b_gate.reshape(1, E)
    al = alpha.reshape(1, 1)
    full = lambda shp: pl.BlockSpec(shp, lambda i: (0,) * len(shp))
    out = pl.pallas_call(
        _fused_body,
        grid=(NB,),
        in_specs=[
            pl.BlockSpec((BT, H), lambda i: (i, 0)),
            full((VD, H)),
            full((E, VD)),
            full((1, E)),
            full((E * P, VD)),
            full((E * I, VD)),
            full((1, E * I)),
            full((E * I, VD)),
            full((E, VD)),
            full((1, 1)),
        ],
        out_specs=pl.BlockSpec((BT, VD), lambda i: (i, 0)),
        out_shape=jax.ShapeDtypeStruct((N, VD), jnp.float32),
        scratch_shapes=[
            pltpu.VMEM((VD, H), bf),
            pltpu.VMEM((E * P + E, VD), bf),
            pltpu.VMEM((E * I, VD), bf),
        ],
        compiler_params=pltpu.CompilerParams(
            dimension_semantics=("arbitrary",)),
    )(x2, Wv, W_gate, bg, Wp, W1f, b1f, W2f, b2, al)
    return out.reshape(B, S, VD)
